# baseline (speedup 1.0000x reference)
import functools

import jax
import jax.numpy as jnp
from jax import lax
from jax.experimental import pallas as pl
from jax.experimental.pallas import tpu as pltpu
from jax.experimental.pallas import tpu_sc as plsc

NUM_KNOTS = 20
X_MIN = -5.0
X_MAX = 5.0
SCALE = (NUM_KNOTS - 1) / (X_MAX - X_MIN)
OFFSET = -X_MIN * SCALE

ROWS, COLS = 200, 16384
NC, NS, L = 2, 16, 16
NW = NC * NS
COLS_W = COLS // NW
CHUNK_C = 128
NCHUNKS = COLS_W // CHUNK_C
VREGS_C = CHUNK_C // L


@functools.partial(
    pl.kernel,
    mesh=plsc.VectorSubcoreMesh(core_axis_name="c", subcore_axis_name="s"),
    out_type=jax.ShapeDtypeStruct((ROWS, COLS), jnp.float32),
    scratch_types=[
        pltpu.VMEM((ROWS, CHUNK_C), jnp.float32),
        pltpu.VMEM((ROWS, CHUNK_C), jnp.float32),
        pltpu.VMEM((ROWS, CHUNK_C), jnp.float32),
        pltpu.VMEM((ROWS, CHUNK_C), jnp.float32),
        pltpu.VMEM((NUM_KNOTS,), jnp.float32),
        pltpu.VMEM((2 * L,), jnp.float32),
        pltpu.VMEM((2 * L,), jnp.float32),
        pltpu.SemaphoreType.DMA,
        pltpu.SemaphoreType.DMA,
        pltpu.SemaphoreType.DMA,
        pltpu.SemaphoreType.DMA,
    ],
    compiler_params=pltpu.CompilerParams(
        needs_layout_passes=False,
        skip_device_barrier=True,
        disable_semaphore_checks=True,
    ),
)
def _interp_sc(x_hbm, values_hbm, out_hbm, in0, in1, out0, out1, vals_v,
               a_v, b_v, si0, si1, so0, so1):
    wid = lax.axis_index("s") * NC + lax.axis_index("c")
    col0 = wid * COLS_W
    ins, outs = [in0, in1], [out0, out1]
    sis, sos = [si0, si1], [so0, so1]

    pltpu.sync_copy(values_hbm, vals_v)

    lane = lax.iota(jnp.int32, L)
    for g in range(2):
        k = lane + g * L
        cur = plsc.load_gather(vals_v, [jnp.minimum(k, NUM_KNOTS - 1)])
        nxt = plsc.load_gather(vals_v, [jnp.minimum(k + 1, NUM_KNOTS - 1)])
        bg = nxt - cur
        b_v[pl.ds(g * L, L)] = bg
        a_v[pl.ds(g * L, L)] = cur - k.astype(jnp.float32) * bg

    b_lo = b_v[pl.ds(0, L)]
    b_hi = b_v[pl.ds(L, L)]

    in_dma = [None] * NCHUNKS
    out_dma = [None] * NCHUNKS
    in_dma[0] = pltpu.async_copy(x_hbm.at[:, pl.ds(col0, CHUNK_C)], in0, si0)

    for k in range(NCHUNKS):
        b = k & 1
        if k + 1 < NCHUNKS:
            in_dma[k + 1] = pltpu.async_copy(
                x_hbm.at[:, pl.ds(col0 + (k + 1) * CHUNK_C, CHUNK_C)],
                ins[1 - b], sis[1 - b])
        in_dma[k].wait()
        if k >= 2:
            out_dma[k - 2].wait()

        ibuf, obuf = ins[b], outs[b]

        @plsc.parallel_loop(0, ROWS, 1, unroll=4)
        def body(r):
            for ci in range(VREGS_C):
                c = ci * L
                xv = ibuf[r, pl.ds(c, L)]
                t = jnp.minimum(jnp.maximum(xv * SCALE + OFFSET, 0.0),
                                float(NUM_KNOTS - 1))
                x0 = t.astype(jnp.int32)
                av = plsc.load_gather(a_v, [x0])
                xm = x0 & (L - 1)
                blo = b_lo.at[xm].get(mode="promise_in_bounds")
                bhi = b_hi.at[xm].get(mode="promise_in_bounds")
                bv = jnp.where(x0 >= L, bhi, blo)
                obuf[r, pl.ds(c, L)] = av + t * bv

        out_dma[k] = pltpu.async_copy(
            obuf, out_hbm.at[:, pl.ds(col0 + k * CHUNK_C, CHUNK_C)], sos[b])

    out_dma[NCHUNKS - 2].wait()
    out_dma[NCHUNKS - 1].wait()


def kernel(x, values):
    out_t = _interp_sc(x.T, values)
    return out_t.T


# speedup vs baseline: 1.0300x; 1.0300x over previous
import functools

import jax
import jax.numpy as jnp
from jax import lax
from jax.experimental import pallas as pl
from jax.experimental.pallas import tpu as pltpu
from jax.experimental.pallas import tpu_sc as plsc

NUM_KNOTS = 20
X_MIN = -5.0
X_MAX = 5.0
SCALE = (NUM_KNOTS - 1) / (X_MAX - X_MIN)
OFFSET = -X_MIN * SCALE

ROWS, COLS = 200, 16384
NC, NS, L = 2, 16, 16
NW = NC * NS
COLS_W = COLS // NW
CHUNK_C = 128
NCHUNKS = COLS_W // CHUNK_C
VREGS_C = CHUNK_C // L


@functools.partial(
    pl.kernel,
    mesh=plsc.VectorSubcoreMesh(core_axis_name="c", subcore_axis_name="s"),
    out_type=jax.ShapeDtypeStruct((ROWS, COLS), jnp.float32),
    scratch_types=[
        pltpu.VMEM((ROWS, CHUNK_C), jnp.float32),
        pltpu.VMEM((ROWS, CHUNK_C), jnp.float32),
        pltpu.VMEM((ROWS, CHUNK_C), jnp.float32),
        pltpu.VMEM((ROWS, CHUNK_C), jnp.float32),
        pltpu.VMEM((NUM_KNOTS,), jnp.float32),
        pltpu.VMEM((2 * L,), jnp.float32),
        pltpu.VMEM((2 * L,), jnp.float32),
        pltpu.SemaphoreType.DMA,
        pltpu.SemaphoreType.DMA,
        pltpu.SemaphoreType.DMA,
        pltpu.SemaphoreType.DMA,
    ],
    compiler_params=pltpu.CompilerParams(
        needs_layout_passes=False,
        skip_device_barrier=True,
        disable_semaphore_checks=True,
    ),
)
def _interp_sc(x_hbm, values_hbm, out_hbm, in0, in1, out0, out1, vals_v,
               a_v, b_v, si0, si1, so0, so1):
    wid = lax.axis_index("s") * NC + lax.axis_index("c")
    col0 = wid * COLS_W
    ins, outs = [in0, in1], [out0, out1]
    sis, sos = [si0, si1], [so0, so1]

    pltpu.sync_copy(values_hbm, vals_v)

    lane = lax.iota(jnp.int32, L)
    for g in range(2):
        k = lane + g * L
        cur = plsc.load_gather(vals_v, [jnp.minimum(k, NUM_KNOTS - 1)])
        nxt = plsc.load_gather(vals_v, [jnp.minimum(k + 1, NUM_KNOTS - 1)])
        bg = nxt - cur
        b_v[pl.ds(g * L, L)] = bg
        a_v[pl.ds(g * L, L)] = cur - k.astype(jnp.float32) * bg

    in_dma = [None] * NCHUNKS
    out_dma = [None] * NCHUNKS
    in_dma[0] = pltpu.async_copy(x_hbm.at[:, pl.ds(col0, CHUNK_C)], in0, si0)

    for k in range(NCHUNKS):
        b = k & 1
        if k + 1 < NCHUNKS:
            in_dma[k + 1] = pltpu.async_copy(
                x_hbm.at[:, pl.ds(col0 + (k + 1) * CHUNK_C, CHUNK_C)],
                ins[1 - b], sis[1 - b])
        in_dma[k].wait()
        if k >= 2:
            out_dma[k - 2].wait()

        ibuf, obuf = ins[b], outs[b]

        @plsc.parallel_loop(0, ROWS, 1, unroll=8)
        def body(r):
            for ci in range(VREGS_C):
                c = ci * L
                xv = ibuf[r, pl.ds(c, L)]
                t = jnp.minimum(jnp.maximum(xv * SCALE + OFFSET, 0.0),
                                float(NUM_KNOTS - 1))
                x0 = t.astype(jnp.int32)
                av = plsc.load_gather(a_v, [x0])
                bv = plsc.load_gather(b_v, [x0])
                obuf[r, pl.ds(c, L)] = av + t * bv

        out_dma[k] = pltpu.async_copy(
            obuf, out_hbm.at[:, pl.ds(col0 + k * CHUNK_C, CHUNK_C)], sos[b])

    out_dma[NCHUNKS - 2].wait()
    out_dma[NCHUNKS - 1].wait()


def kernel(x, values):
    out_t = _interp_sc(x.T, values)
    return out_t.T


# speedup vs baseline: 1.1026x; 1.0704x over previous
import functools

import jax
import jax.numpy as jnp
from jax import lax
from jax.experimental import pallas as pl
from jax.experimental.pallas import tpu as pltpu
from jax.experimental.pallas import tpu_sc as plsc

NUM_KNOTS = 20
X_MIN = -5.0
X_MAX = 5.0
SCALE = (NUM_KNOTS - 1) / (X_MAX - X_MIN)
OFFSET = -X_MIN * SCALE

ROWS, COLS = 200, 16384
NC, NS, L = 2, 16, 16
NW = NC * NS
COLS_W = COLS // NW
CHUNK_C = 128
NCHUNKS = COLS_W // CHUNK_C
VREGS_C = CHUNK_C // L


@functools.partial(
    pl.kernel,
    mesh=plsc.VectorSubcoreMesh(core_axis_name="c", subcore_axis_name="s"),
    out_type=jax.ShapeDtypeStruct((ROWS, COLS), jnp.float32),
    scratch_types=[
        pltpu.VMEM((ROWS, CHUNK_C), jnp.float32),
        pltpu.VMEM((ROWS, CHUNK_C), jnp.float32),
        pltpu.VMEM((ROWS, CHUNK_C), jnp.float32),
        pltpu.VMEM((ROWS, CHUNK_C), jnp.float32),
        pltpu.VMEM((NUM_KNOTS,), jnp.float32),
        pltpu.VMEM((2 * L,), jnp.float32),
        pltpu.VMEM((2 * L,), jnp.float32),
        pltpu.SemaphoreType.DMA,
        pltpu.SemaphoreType.DMA,
        pltpu.SemaphoreType.DMA,
        pltpu.SemaphoreType.DMA,
    ],
    compiler_params=pltpu.CompilerParams(
        needs_layout_passes=False,
        skip_device_barrier=True,
        disable_semaphore_checks=True,
    ),
)
def _interp_sc(x_hbm, values_hbm, out_hbm, in0, in1, out0, out1, vals_v,
               a_v, b_v, si0, si1, so0, so1):
    wid = lax.axis_index("s") * NC + lax.axis_index("c")
    col0 = wid * COLS_W
    ins, outs = [in0, in1], [out0, out1]
    sis, sos = [si0, si1], [so0, so1]

    pltpu.sync_copy(values_hbm, vals_v)

    lane = lax.iota(jnp.int32, L)
    for g in range(2):
        k = lane + g * L
        cur = plsc.load_gather(vals_v, [jnp.minimum(k, NUM_KNOTS - 1)])
        nxt = plsc.load_gather(vals_v, [jnp.minimum(k + 1, NUM_KNOTS - 1)])
        bg = nxt - cur
        b_v[pl.ds(g * L, L)] = bg
        a_v[pl.ds(g * L, L)] = cur - k.astype(jnp.float32) * bg

    in_dma = [None] * NCHUNKS
    out_dma = [None] * NCHUNKS
    in_dma[0] = pltpu.async_copy(x_hbm.at[:, pl.ds(col0, CHUNK_C)], in0, si0)

    for k in range(NCHUNKS):
        b = k & 1
        if k + 1 < NCHUNKS:
            in_dma[k + 1] = pltpu.async_copy(
                x_hbm.at[:, pl.ds(col0 + (k + 1) * CHUNK_C, CHUNK_C)],
                ins[1 - b], sis[1 - b])
        in_dma[k].wait()
        if k >= 2:
            out_dma[k - 2].wait()

        ibuf, obuf = ins[b], outs[b]

        @plsc.parallel_loop(0, ROWS, 1, unroll=2)
        def body(r):
            for ci in range(VREGS_C):
                c = ci * L
                xv = ibuf[r, pl.ds(c, L)]
                t = jnp.minimum(jnp.maximum(xv * SCALE + OFFSET, 0.0),
                                float(NUM_KNOTS - 1))
                x0 = t.astype(jnp.int32)
                av = plsc.load_gather(a_v, [x0])
                bv = plsc.load_gather(b_v, [x0])
                obuf[r, pl.ds(c, L)] = av + t * bv

        out_dma[k] = pltpu.async_copy(
            obuf, out_hbm.at[:, pl.ds(col0 + k * CHUNK_C, CHUNK_C)], sos[b])

    out_dma[NCHUNKS - 2].wait()
    out_dma[NCHUNKS - 1].wait()


def kernel(x, values):
    out_t = _interp_sc(x.T, values)
    return out_t.T


# speedup vs baseline: 1.1528x; 1.0455x over previous
import functools

import jax
import jax.numpy as jnp
from jax import lax
from jax.experimental import pallas as pl
from jax.experimental.pallas import tpu as pltpu
from jax.experimental.pallas import tpu_sc as plsc

NUM_KNOTS = 20
X_MIN = -5.0
X_MAX = 5.0
SCALE = (NUM_KNOTS - 1) / (X_MAX - X_MIN)
OFFSET = -X_MIN * SCALE

ROWS, COLS = 200, 16384
NC, NS, L = 2, 16, 16
NW = NC * NS
COLS_W = COLS // NW
CHUNK_C = 128
NCHUNKS = COLS_W // CHUNK_C
VREGS_C = CHUNK_C // L


@functools.partial(
    pl.kernel,
    mesh=plsc.VectorSubcoreMesh(core_axis_name="c", subcore_axis_name="s"),
    out_type=jax.ShapeDtypeStruct((ROWS, COLS), jnp.float32),
    scratch_types=[
        pltpu.VMEM((ROWS, CHUNK_C), jnp.float32),
        pltpu.VMEM((ROWS, CHUNK_C), jnp.float32),
        pltpu.VMEM((ROWS, CHUNK_C), jnp.float32),
        pltpu.VMEM((ROWS, CHUNK_C), jnp.float32),
        pltpu.VMEM((NUM_KNOTS,), jnp.float32),
        pltpu.VMEM((2 * L,), jnp.float32),
        pltpu.VMEM((2 * L,), jnp.float32),
        pltpu.SemaphoreType.DMA,
        pltpu.SemaphoreType.DMA,
        pltpu.SemaphoreType.DMA,
        pltpu.SemaphoreType.DMA,
    ],
    compiler_params=pltpu.CompilerParams(
        needs_layout_passes=False,
        skip_device_barrier=True,
        disable_semaphore_checks=True,
    ),
)
def _interp_sc(x_hbm, values_hbm, out_hbm, in0, in1, out0, out1, vals_v,
               a_v, b_v, si0, si1, so0, so1):
    wid = lax.axis_index("s") * NC + lax.axis_index("c")
    col0 = wid * COLS_W
    ins, outs = [in0, in1], [out0, out1]
    sis, sos = [si0, si1], [so0, so1]

    in_dma = [None] * NCHUNKS
    out_dma = [None] * NCHUNKS
    in_dma[0] = pltpu.async_copy(x_hbm.at[:, pl.ds(col0, CHUNK_C)], in0, si0)

    pltpu.sync_copy(values_hbm, vals_v)

    lane = lax.iota(jnp.int32, L)
    for g in range(2):
        k = lane + g * L
        cur = plsc.load_gather(vals_v, [jnp.minimum(k, NUM_KNOTS - 1)])
        nxt = plsc.load_gather(vals_v, [jnp.minimum(k + 1, NUM_KNOTS - 1)])
        bg = nxt - cur
        b_v[pl.ds(g * L, L)] = bg
        a_v[pl.ds(g * L, L)] = cur - k.astype(jnp.float32) * bg

    for k in range(NCHUNKS):
        b = k & 1
        if k + 1 < NCHUNKS:
            in_dma[k + 1] = pltpu.async_copy(
                x_hbm.at[:, pl.ds(col0 + (k + 1) * CHUNK_C, CHUNK_C)],
                ins[1 - b], sis[1 - b])
        in_dma[k].wait()
        if k >= 2:
            out_dma[k - 2].wait()

        ibuf, obuf = ins[b], outs[b]

        @plsc.parallel_loop(0, ROWS, 1, unroll=2)
        def body(r):
            for ci in range(VREGS_C):
                c = ci * L
                xv = ibuf[r, pl.ds(c, L)]
                t = jnp.minimum(jnp.maximum(xv * SCALE + OFFSET, 0.0),
                                float(NUM_KNOTS - 1))
                x0 = t.astype(jnp.int32)
                av = plsc.load_gather(a_v, [x0])
                bv = plsc.load_gather(b_v, [x0])
                obuf[r, pl.ds(c, L)] = av + t * bv

        out_dma[k] = pltpu.async_copy(
            obuf, out_hbm.at[:, pl.ds(col0 + k * CHUNK_C, CHUNK_C)], sos[b])

    out_dma[NCHUNKS - 2].wait()
    out_dma[NCHUNKS - 1].wait()


def kernel(x, values):
    out_t = _interp_sc(x.T, values)
    return out_t.T
